# Initial kernel scaffold; baseline (speedup 1.0000x reference)
import jax
import jax.numpy as jnp
from jax.experimental import pallas as pl


def kernel(ego_embed, edge_index, edge_type, relation_embed, dropout):
    raise NotImplementedError("write your pallas kernel here")


# speedup vs baseline: 1.1187x; 1.1187x over previous
import jax
import jax.numpy as jnp
from jax.experimental import pallas as pl

N_NODES = 10000
D = 128
N_HOPS = 2
_ROWS = 1000


def _norm_body(s_ref, r_ref, n_ref, o_ref):
    s = s_ref[...]
    norm = jnp.sqrt(jnp.sum(s * s, axis=1, keepdims=True))
    n = s / jnp.maximum(norm, 1e-12)
    n_ref[...] = n
    o_ref[...] = r_ref[...] + n


def _normalize_and_accum(s, res):
    grid = (s.shape[0] // _ROWS,)
    return pl.pallas_call(
        _norm_body,
        grid=grid,
        in_specs=[
            pl.BlockSpec((_ROWS, D), lambda i: (i, 0)),
            pl.BlockSpec((_ROWS, D), lambda i: (i, 0)),
        ],
        out_specs=[
            pl.BlockSpec((_ROWS, D), lambda i: (i, 0)),
            pl.BlockSpec((_ROWS, D), lambda i: (i, 0)),
        ],
        out_shape=[
            jax.ShapeDtypeStruct(s.shape, s.dtype),
            jax.ShapeDtypeStruct(s.shape, s.dtype),
        ],
    )(s, res)


def kernel(ego_embed, edge_index, edge_type, relation_embed, dropout):
    head = edge_index[0].astype(jnp.int32)
    tail = edge_index[1].astype(jnp.int32)
    etype = edge_type.astype(jnp.int32)
    res = ego_embed
    ego = ego_embed
    for _ in range(N_HOPS):
        neigh = ego[tail] * relation_embed[etype]
        agg = jax.ops.segment_sum(neigh, head, num_segments=N_NODES)
        ego, res = _normalize_and_accum(agg, res)
    return res


# speedup vs baseline: 2.4859x; 2.2221x over previous
import functools

import jax
import jax.numpy as jnp
from jax import lax
from jax.experimental import pallas as pl
from jax.experimental.pallas import tpu as pltpu
from jax.experimental.pallas import tpu_sc as plsc

N_NODES = 10000
D = 128
N_REL = 16
N_HOPS = 2

NC = 2
NS = 16
NW = NC * NS
C = 128
NCHUNK = 80
EPW_PAD = NCHUNK * C
ACC_ROWS = 10240
TRASH = N_NODES
ZPT = ACC_ROWS // NS


def _agg_body(ent_hbm, rel_hbm, tails_hbm, types_hbm, heads_hbm, out_hbm,
              acc, rel_v, rows0, rows1,
              t0, t1, t2, t3, y0, y1, y2, y3, h0, h1, h2, h3,
              gsem0, gsem1, ssem0, ssem1, isem0, isem1, isem2, isem3):
    cid = lax.axis_index("c")
    sid = lax.axis_index("s")
    wid = sid * NC + cid

    rows = (rows0, rows1)
    tails = (t0, t1, t2, t3)
    types = (y0, y1, y2, y3)
    heads = (h0, h1, h2, h3)
    gsem = (gsem0, gsem1)
    ssem = (ssem0, ssem1)
    isem = (isem0, isem1, isem2, isem3)

    def _zrow(r, carry):
        for c8 in range(8):
            rows0[r, pl.ds(c8 * 16, 16)] = jnp.zeros((16,), jnp.float32)
        return carry
    lax.fori_loop(0, C, _zrow, 0)
    for j in range(ZPT // C):
        pltpu.sync_copy(rows0, acc.at[pl.ds(sid * ZPT + j * C, C)])

    pltpu.sync_copy(rel_hbm, rel_v)

    plsc.subcore_barrier()

    def _start_idx(g, s):
        pltpu.async_copy(tails_hbm.at[wid, g], tails[s], isem[s])
        pltpu.async_copy(types_hbm.at[wid, g], types[s], isem[s])
        pltpu.async_copy(heads_hbm.at[wid, g], heads[s], isem[s])

    def _wait_idx(g, s):
        pltpu.make_async_copy(tails_hbm.at[wid, g], tails[s], isem[s]).wait()
        pltpu.make_async_copy(types_hbm.at[wid, g], types[s], isem[s]).wait()
        pltpu.make_async_copy(heads_hbm.at[wid, g], heads[s], isem[s]).wait()

    def _start_gather(s, b):
        pltpu.async_copy(ent_hbm.at[tails[s]], rows[b], gsem[b])

    def _wait_gather(s, b):
        pltpu.make_async_copy(ent_hbm.at[tails[s]], rows[b], gsem[b]).wait()

    def _start_scatter(s, b):
        pltpu.async_copy(rows[b], acc.at[heads[s]], ssem[b], add=True)

    def _wait_scatter(s, b):
        pltpu.make_async_copy(rows[b], acc.at[heads[s]], ssem[b]).wait()

    iota16 = lax.iota(jnp.int32, 16)

    def _multiply(s, b):
        rb = rows[b]
        tv = types[s]

        def _row(r, carry):
            t16 = plsc.load_gather(tv, [jnp.full((16,), 0, jnp.int32) + r])
            for c8 in range(8):
                sl = pl.ds(c8 * 16, 16)
                rl = plsc.load_gather(rel_v, [t16, iota16 + (c8 * 16)])
                rb[r, sl] = rb[r, sl] * rl
            return carry
        lax.fori_loop(0, C, _row, 0)

    _start_idx(0, 0)
    _start_idx(1, 1)
    _wait_idx(0, 0)
    _start_gather(0, 0)

    def _step(t, carry):
        for k in range(4):
            g = 4 * t + k
            b = k % 2

            if k == 0:
                @pl.when(t > 0)
                def _():
                    _wait_scatter(3, 1 - b)
            else:
                _wait_scatter(k - 1, 1 - b)

            if k < 2:
                _start_idx(g + 2, k + 2)
            else:
                @pl.when(t < (NCHUNK // 4) - 1)
                def _():
                    _start_idx(g + 2, k - 2)

            if k < 3:
                _wait_idx(g + 1, k + 1)
                _start_gather(k + 1, 1 - b)
            else:
                @pl.when(t < (NCHUNK // 4) - 1)
                def _():
                    _wait_idx(g + 1, 0)
                    _start_gather(0, 1 - b)

            _wait_gather(k, b)
            _multiply(k, b)
            _start_scatter(k, b)
        return carry

    lax.fori_loop(0, NCHUNK // 4, _step, 0)
    _wait_scatter(3, 1)

    plsc.subcore_barrier()

    for j in range(ZPT // C):
        base = sid * ZPT + j * C
        pltpu.sync_copy(acc.at[pl.ds(base, C)], rows0)
        pltpu.sync_copy(rows0, out_hbm.at[cid, pl.ds(base, C)])


@jax.jit
def _sc_aggregate(ent, rel, tails, types, heads):
    mesh = plsc.VectorSubcoreMesh(core_axis_name="c", subcore_axis_name="s")
    f = pl.kernel(
        _agg_body,
        out_type=jax.ShapeDtypeStruct((NC, ACC_ROWS, D), jnp.float32),
        mesh=mesh,
        compiler_params=pltpu.CompilerParams(
            use_tc_tiling_on_sc=False, needs_layout_passes=False
        ),
        scratch_types=[
            pltpu.VMEM_SHARED((ACC_ROWS, D), jnp.float32),
            pltpu.VMEM((N_REL, D), jnp.float32),
            pltpu.VMEM((C, D), jnp.float32),
            pltpu.VMEM((C, D), jnp.float32),
        ]
        + [pltpu.VMEM((C,), jnp.int32)] * 12
        + [pltpu.SemaphoreType.DMA] * 8,
    )
    return f(ent, rel, tails, types, heads)


_ROWS = 1000


def _comb_body(p_ref, r_ref, n_ref, o_ref):
    s = p_ref[0] + p_ref[1]
    norm = jnp.sqrt(jnp.sum(s * s, axis=1, keepdims=True))
    n = s / jnp.maximum(norm, 1e-12)
    n_ref[...] = n
    o_ref[...] = r_ref[...] + n


def _combine_normalize(parts, res):
    grid = (N_NODES // _ROWS,)
    return pl.pallas_call(
        _comb_body,
        grid=grid,
        in_specs=[
            pl.BlockSpec((NC, _ROWS, D), lambda i: (0, i, 0)),
            pl.BlockSpec((_ROWS, D), lambda i: (i, 0)),
        ],
        out_specs=[
            pl.BlockSpec((_ROWS, D), lambda i: (i, 0)),
            pl.BlockSpec((_ROWS, D), lambda i: (i, 0)),
        ],
        out_shape=[
            jax.ShapeDtypeStruct((N_NODES, D), jnp.float32),
            jax.ShapeDtypeStruct((N_NODES, D), jnp.float32),
        ],
    )(parts, res)


def kernel(ego_embed, edge_index, edge_type, relation_embed, dropout):
    n_edges = edge_index.shape[1]
    epw = n_edges // NW
    pad = EPW_PAD - epw

    def _prep(x, fill):
        x = x.astype(jnp.int32).reshape(NW, epw)
        x = jnp.pad(x, ((0, 0), (0, pad)), constant_values=fill)
        return x.reshape(NW, NCHUNK, C)

    heads = _prep(edge_index[0], TRASH)
    tails = _prep(edge_index[1], 0)
    types = _prep(edge_type, 0)

    res = ego_embed
    ego = ego_embed
    for _ in range(N_HOPS):
        parts = _sc_aggregate(ego, relation_embed, tails, types, heads)
        ego, res = _combine_normalize(parts, res)
    return res


# speedup vs baseline: 3.8836x; 1.5622x over previous
import jax
import jax.numpy as jnp
from jax import lax
from jax.experimental import pallas as pl
from jax.experimental.pallas import tpu as pltpu
from jax.experimental.pallas import tpu_sc as plsc

N_NODES = 10000
D = 128
N_REL = 16
N_HOPS = 2

NC = 2
NS = 16
NW = NC * NS
C = 128
G = 8
NGRP = 10
NCHUNK = NGRP * G
EPW_PAD = NCHUNK * C
ACC_ROWS = 10240
TRASH = N_NODES
ZPT = ACC_ROWS // NS


def _agg_body(entx_hbm, fused_hbm, heads_hbm, out_hbm,
              acc, rows0, rows1, f0, f1, h0, h1,
              gsem0, gsem1, ssem0, ssem1, isem0, isem1):
    cid = lax.axis_index("c")
    sid = lax.axis_index("s")
    wid = sid * NC + cid

    rows = (rows0, rows1)
    fgrp = (f0, f1)
    hgrp = (h0, h1)
    gsem = (gsem0, gsem1)
    ssem = (ssem0, ssem1)
    isem = (isem0, isem1)

    def _zrow(r, carry):
        for c8 in range(8):
            rows0[r, pl.ds(c8 * 16, 16)] = jnp.zeros((16,), jnp.float32)
        return carry
    lax.fori_loop(0, C, _zrow, 0)
    for j in range(ZPT // C):
        pltpu.sync_copy(rows0, acc.at[pl.ds(sid * ZPT + j * C, C)])

    plsc.subcore_barrier()

    def _start_idx(q, p):
        pltpu.async_copy(fused_hbm.at[wid, q], fgrp[p], isem[p])
        pltpu.async_copy(heads_hbm.at[wid, q], hgrp[p], isem[p])

    def _wait_idx(q, p):
        pltpu.make_async_copy(fused_hbm.at[wid, q], fgrp[p], isem[p]).wait()
        pltpu.make_async_copy(heads_hbm.at[wid, q], hgrp[p], isem[p]).wait()

    def _wait_scatter(j, p, b):
        pltpu.make_async_copy(rows[b], acc.at[hgrp[p].at[j]], ssem[b]).wait()

    _start_idx(0, 0)

    def _step(q2, carry):
        for p in range(2):
            q = 2 * q2 + p
            _wait_idx(q, p)
            for j in range(G):
                b = j % 2
                if j == 2:
                    if p == 0:
                        _start_idx(q + 1, 1)
                    else:
                        @pl.when(q2 < 4)
                        def _():
                            _start_idx(q + 1, 0)
                if p == 0 and j < 2:
                    @pl.when(q2 > 0)
                    def _():
                        _wait_scatter(G - 2 + j, 1, b)
                elif j < 2:
                    _wait_scatter(G - 2 + j, 0, b)
                else:
                    _wait_scatter(j - 2, p, b)
                pltpu.async_copy(entx_hbm.at[fgrp[p].at[j]], rows[b], gsem[b])
                pltpu.make_async_copy(
                    entx_hbm.at[fgrp[p].at[j]], rows[b], gsem[b]).wait()
                pltpu.async_copy(rows[b], acc.at[hgrp[p].at[j]], ssem[b],
                                 add=True)
        return carry

    lax.fori_loop(0, NGRP // 2, _step, 0)
    _wait_scatter(G - 2, 1, 0)
    _wait_scatter(G - 1, 1, 1)

    plsc.subcore_barrier()

    for j in range(ZPT // C):
        base = sid * ZPT + j * C
        pltpu.sync_copy(acc.at[pl.ds(base, C)], rows0)
        pltpu.sync_copy(rows0, out_hbm.at[cid, pl.ds(base, C)])


@jax.jit
def _sc_aggregate(entx, fused, heads):
    mesh = plsc.VectorSubcoreMesh(core_axis_name="c", subcore_axis_name="s")
    f = pl.kernel(
        _agg_body,
        out_type=jax.ShapeDtypeStruct((NC, ACC_ROWS, D), jnp.float32),
        mesh=mesh,
        compiler_params=pltpu.CompilerParams(needs_layout_passes=False),
        scratch_types=[
            pltpu.VMEM_SHARED((ACC_ROWS, D), jnp.float32),
            pltpu.VMEM((C, D), jnp.float32),
            pltpu.VMEM((C, D), jnp.float32),
            pltpu.VMEM((G, C), jnp.int32),
            pltpu.VMEM((G, C), jnp.int32),
            pltpu.VMEM((G, C), jnp.int32),
            pltpu.VMEM((G, C), jnp.int32),
        ]
        + [pltpu.SemaphoreType.DMA] * 6,
    )
    return f(entx, fused, heads)


_ROWS = 1000


def _entx_body(e_ref, rl_ref, o_ref):
    r = pl.program_id(0)
    o_ref[...] = e_ref[...] * rl_ref[r, :][None, :]


def _build_entx(ent, rel):
    return pl.pallas_call(
        _entx_body,
        grid=(N_REL, N_NODES // _ROWS),
        in_specs=[
            pl.BlockSpec((_ROWS, D), lambda r, i: (i, 0)),
            pl.BlockSpec((N_REL, D), lambda r, i: (0, 0)),
        ],
        out_specs=pl.BlockSpec((_ROWS, D), lambda r, i: (r * (N_NODES // _ROWS) + i, 0)),
        out_shape=jax.ShapeDtypeStruct((N_REL * N_NODES, D), jnp.float32),
    )(ent, rel)


def _comb_body(p_ref, r_ref, n_ref, o_ref):
    s = p_ref[0] + p_ref[1]
    norm = jnp.sqrt(jnp.sum(s * s, axis=1, keepdims=True))
    n = s / jnp.maximum(norm, 1e-12)
    n_ref[...] = n
    o_ref[...] = r_ref[...] + n


def _combine_normalize(parts, res):
    grid = (N_NODES // _ROWS,)
    return pl.pallas_call(
        _comb_body,
        grid=grid,
        in_specs=[
            pl.BlockSpec((NC, _ROWS, D), lambda i: (0, i, 0)),
            pl.BlockSpec((_ROWS, D), lambda i: (i, 0)),
        ],
        out_specs=[
            pl.BlockSpec((_ROWS, D), lambda i: (i, 0)),
            pl.BlockSpec((_ROWS, D), lambda i: (i, 0)),
        ],
        out_shape=[
            jax.ShapeDtypeStruct((N_NODES, D), jnp.float32),
            jax.ShapeDtypeStruct((N_NODES, D), jnp.float32),
        ],
    )(parts, res)


def kernel(ego_embed, edge_index, edge_type, relation_embed, dropout):
    n_edges = edge_index.shape[1]
    epw = n_edges // NW
    pad = EPW_PAD - epw

    def _prep(x, fill):
        x = x.astype(jnp.int32).reshape(NW, epw)
        x = jnp.pad(x, ((0, 0), (0, pad)), constant_values=fill)
        return x.reshape(NW, NGRP, G, C)

    heads = _prep(edge_index[0], TRASH)
    tails = edge_index[1].astype(jnp.int32)
    types = edge_type.astype(jnp.int32)
    fused = _prep(types * N_NODES + tails, 0)

    res = ego_embed
    ego = ego_embed
    for _ in range(N_HOPS):
        entx = _build_entx(ego, relation_embed)
        parts = _sc_aggregate(entx, fused, heads)
        ego, res = _combine_normalize(parts, res)
    return res


# speedup vs baseline: 4.3644x; 1.1238x over previous
import jax
import jax.numpy as jnp
from jax import lax
from jax.experimental import pallas as pl
from jax.experimental.pallas import tpu as pltpu
from jax.experimental.pallas import tpu_sc as plsc

N_NODES = 10000
D = 128
N_REL = 16
N_HOPS = 2

NC = 2
NS = 16
NW = NC * NS
C = 128
G = 8
NGRP = 10
NCHUNK = NGRP * G
EPW_PAD = NCHUNK * C
ACC_ROWS = 10240
TRASH = N_NODES
ZPT = ACC_ROWS // NS


def _agg_body(entx_hbm, fused_hbm, heads_hbm, out_hbm,
              acc, rows0, rows1, f0, f1, h0, h1,
              gsem0, gsem1, ssem0, ssem1, isem0, isem1):
    cid = lax.axis_index("c")
    sid = lax.axis_index("s")
    wid = sid * NC + cid

    rows = (rows0, rows1)
    fgrp = (f0, f1)
    hgrp = (h0, h1)
    gsem = (gsem0, gsem1)
    ssem = (ssem0, ssem1)
    isem = (isem0, isem1)

    def _zrow(r, carry):
        for c8 in range(8):
            rows0[r, pl.ds(c8 * 16, 16)] = jnp.zeros((16,), jnp.float32)
        return carry
    lax.fori_loop(0, C, _zrow, 0)
    for j in range(ZPT // C):
        pltpu.sync_copy(rows0, acc.at[pl.ds(sid * ZPT + j * C, C)])

    plsc.subcore_barrier()

    def _start_idx(q, p):
        pltpu.async_copy(fused_hbm.at[wid, q], fgrp[p], isem[p])
        pltpu.async_copy(heads_hbm.at[wid, q], hgrp[p], isem[p])

    def _wait_idx(q, p):
        pltpu.make_async_copy(fused_hbm.at[wid, q], fgrp[p], isem[p]).wait()
        pltpu.make_async_copy(heads_hbm.at[wid, q], hgrp[p], isem[p]).wait()

    def _wait_scatter(j, p, b):
        pltpu.make_async_copy(rows[b], acc.at[hgrp[p].at[j]], ssem[b]).wait()

    def _start_gather(idx_ref, b):
        pltpu.async_copy(entx_hbm.at[idx_ref], rows[b], gsem[b])

    def _wait_gather(idx_ref, b):
        pltpu.make_async_copy(entx_hbm.at[idx_ref], rows[b], gsem[b]).wait()

    _start_idx(0, 0)
    _wait_idx(0, 0)
    _start_gather(fgrp[0].at[0], 0)

    def _step(q2, carry):
        for p in range(2):
            q = 2 * q2 + p
            for j in range(G):
                b = j % 2
                if j == 2:
                    if p == 0:
                        _start_idx(q + 1, 1)
                    else:
                        @pl.when(q2 < 4)
                        def _():
                            _start_idx(q + 1, 0)
                if p == 0 and j == 0:
                    @pl.when(q2 > 0)
                    def _():
                        _wait_scatter(G - 1, 1, 1 - b)
                elif j == 0:
                    _wait_scatter(G - 1, 0, 1 - b)
                else:
                    _wait_scatter(j - 1, p, 1 - b)
                if j < G - 1:
                    _start_gather(fgrp[p].at[j + 1], 1 - b)
                elif p == 0:
                    _wait_idx(q + 1, 1)
                    _start_gather(fgrp[1].at[0], 1 - b)
                else:
                    @pl.when(q2 < 4)
                    def _():
                        _wait_idx(q + 1, 0)
                        _start_gather(fgrp[0].at[0], 1 - b)
                _wait_gather(fgrp[p].at[j], b)
                pltpu.async_copy(rows[b], acc.at[hgrp[p].at[j]], ssem[b],
                                 add=True)
        return carry

    lax.fori_loop(0, NGRP // 2, _step, 0)
    _wait_scatter(G - 1, 1, 1)

    plsc.subcore_barrier()

    for j in range(ZPT // C):
        base = sid * ZPT + j * C
        pltpu.sync_copy(acc.at[pl.ds(base, C)], rows0)
        pltpu.sync_copy(rows0, out_hbm.at[cid, pl.ds(base, C)])


@jax.jit
def _sc_aggregate(entx, fused, heads):
    mesh = plsc.VectorSubcoreMesh(core_axis_name="c", subcore_axis_name="s")
    f = pl.kernel(
        _agg_body,
        out_type=jax.ShapeDtypeStruct((NC, ACC_ROWS, D), jnp.float32),
        mesh=mesh,
        compiler_params=pltpu.CompilerParams(needs_layout_passes=False),
        scratch_types=[
            pltpu.VMEM_SHARED((ACC_ROWS, D), jnp.float32),
            pltpu.VMEM((C, D), jnp.float32),
            pltpu.VMEM((C, D), jnp.float32),
            pltpu.VMEM((G, C), jnp.int32),
            pltpu.VMEM((G, C), jnp.int32),
            pltpu.VMEM((G, C), jnp.int32),
            pltpu.VMEM((G, C), jnp.int32),
        ]
        + [pltpu.SemaphoreType.DMA] * 6,
    )
    return f(entx, fused, heads)


_ROWS = 1000


def _entx_body(e_ref, rl_ref, o_ref):
    r = pl.program_id(1)
    o_ref[...] = e_ref[...] * rl_ref[r, :][None, :]


def _build_entx(ent, rel):
    return pl.pallas_call(
        _entx_body,
        grid=(N_NODES // _ROWS, N_REL),
        in_specs=[
            pl.BlockSpec((_ROWS, D), lambda i, r: (i, 0)),
            pl.BlockSpec((N_REL, D), lambda i, r: (0, 0)),
        ],
        out_specs=pl.BlockSpec(
            (_ROWS, D), lambda i, r: (r * (N_NODES // _ROWS) + i, 0)),
        out_shape=jax.ShapeDtypeStruct((N_REL * N_NODES, D), jnp.float32),
    )(ent, rel)


def _comb_body(p_ref, r_ref, n_ref, o_ref):
    s = p_ref[0] + p_ref[1]
    norm = jnp.sqrt(jnp.sum(s * s, axis=1, keepdims=True))
    n = s / jnp.maximum(norm, 1e-12)
    n_ref[...] = n
    o_ref[...] = r_ref[...] + n


def _combine_normalize(parts, res):
    grid = (N_NODES // _ROWS,)
    return pl.pallas_call(
        _comb_body,
        grid=grid,
        in_specs=[
            pl.BlockSpec((NC, _ROWS, D), lambda i: (0, i, 0)),
            pl.BlockSpec((_ROWS, D), lambda i: (i, 0)),
        ],
        out_specs=[
            pl.BlockSpec((_ROWS, D), lambda i: (i, 0)),
            pl.BlockSpec((_ROWS, D), lambda i: (i, 0)),
        ],
        out_shape=[
            jax.ShapeDtypeStruct((N_NODES, D), jnp.float32),
            jax.ShapeDtypeStruct((N_NODES, D), jnp.float32),
        ],
    )(parts, res)


def kernel(ego_embed, edge_index, edge_type, relation_embed, dropout):
    n_edges = edge_index.shape[1]
    epw = n_edges // NW
    pad = EPW_PAD - epw

    def _prep(x, fill):
        x = x.astype(jnp.int32).reshape(NW, epw)
        x = jnp.pad(x, ((0, 0), (0, pad)), constant_values=fill)
        return x.reshape(NW, NGRP, G, C)

    heads = _prep(edge_index[0], TRASH)
    tails = edge_index[1].astype(jnp.int32)
    types = edge_type.astype(jnp.int32)
    fused = _prep(types * N_NODES + tails, 0)

    res = ego_embed
    ego = ego_embed
    for _ in range(N_HOPS):
        entx = _build_entx(ego, relation_embed)
        parts = _sc_aggregate(entx, fused, heads)
        ego, res = _combine_normalize(parts, res)
    return res
